# baseline (speedup 1.0000x reference)
import functools

import jax
import jax.numpy as jnp
from jax import lax
from jax.experimental import pallas as pl
from jax.experimental.pallas import tpu as pltpu
from jax.experimental.pallas import tpu_sc as plsc

_MV_CHUNK = 16384


def _matvec_body(tab_ref, w1_ref, w2_ref, out_ref):
    v = jnp.dot(w1_ref[...], w2_ref[...], preferred_element_type=jnp.float32)
    acc = lax.dot_general(v.T, tab_ref[...], (((1,), (1,)), ((), ())),
                          preferred_element_type=jnp.float32)
    out_ref[...] = acc[0]


def _head_body(t_ref, w3_ref, b1_ref, w2_ref, b2_ref, b3_ref, out_ref):
    c = jnp.dot(b1_ref[...], w2_ref[...], preferred_element_type=jnp.float32)
    const = (c[0, 0] + b2_ref[0, 0]) * jnp.sum(w3_ref[...]) + b3_ref[0, 0]
    acc = jnp.sum(t_ref[...], axis=1, keepdims=True)
    out_ref[...] = jax.nn.sigmoid(acc + const)


def _make_gather_fc3(num_workers, seqs_per_w, seq_len, w3_pad):
    nc = plsc.get_sparse_core_info().num_cores
    mesh = plsc.VectorSubcoreMesh(core_axis_name="c", subcore_axis_name="s")
    n_full = seq_len // 16
    n_sl = n_full + 1

    @functools.partial(
        pl.kernel,
        mesh=mesh,
        out_type=jax.ShapeDtypeStruct((num_workers * seqs_per_w, 16),
                                      jnp.float32),
        scratch_types=[
            pltpu.VMEM((seqs_per_w, w3_pad), jnp.int32),
            pltpu.VMEM((seqs_per_w, w3_pad), jnp.float32),
            pltpu.VMEM((seqs_per_w, 16), jnp.float32),
            pltpu.VMEM((w3_pad,), jnp.float32),
            pltpu.SemaphoreType.DMA,
        ],
    )
    def gather_kernel(ids_hbm, p_hbm, w3_hbm, out_hbm,
                      idx_v, val_v, acc_v, w3_v, sem):
        wid = lax.axis_index("s") * nc + lax.axis_index("c")
        base = pl.multiple_of(wid * seqs_per_w, 8)
        pltpu.sync_copy(ids_hbm.at[pl.ds(base, seqs_per_w), :], idx_v)
        pltpu.sync_copy(w3_hbm, w3_v)

        def fire(j, carry):
            pltpu.async_copy(p_hbm.at[idx_v.at[j, pl.ds(0, n_full * 16)]],
                             val_v.at[j, pl.ds(0, n_full * 16)], sem)
            pltpu.async_copy(p_hbm.at[idx_v.at[j, pl.ds(n_full * 16, 16)]],
                             val_v.at[j, pl.ds(n_full * 16, 16)], sem)
            return carry

        lax.fori_loop(0, seqs_per_w, fire, 0, unroll=False)

        def drain_compute(j, carry):
            pltpu.make_async_copy(
                p_hbm.at[idx_v.at[j, pl.ds(0, n_full * 16)]],
                val_v.at[j, pl.ds(0, n_full * 16)], sem).wait()
            pltpu.make_async_copy(
                p_hbm.at[idx_v.at[j, pl.ds(n_full * 16, 16)]],
                val_v.at[j, pl.ds(n_full * 16, 16)], sem).wait()
            acc = jnp.zeros((16,), jnp.float32)
            for g in range(n_sl):
                acc = acc + (val_v[j, pl.ds(g * 16, 16)]
                             * w3_v[pl.ds(g * 16, 16)])
            acc_v[j, :] = acc
            return carry

        lax.fori_loop(0, seqs_per_w, drain_compute, 0, unroll=False)
        pltpu.sync_copy(acc_v, out_hbm.at[pl.ds(base, seqs_per_w), :])

    return gather_kernel


def kernel(input_ids, emb_table, W1, b1, W2, b2, W3, b3):
    B, S = input_ids.shape
    V, D = emb_table.shape
    H = W1.shape[1]

    grid = (V + _MV_CHUNK - 1) // _MV_CHUNK
    p = pl.pallas_call(
        _matvec_body,
        grid=(grid,),
        in_specs=[
            pl.BlockSpec((_MV_CHUNK, D), lambda i: (i, 0)),
            pl.BlockSpec((D, H), lambda i: (0, 0)),
            pl.BlockSpec((H, 1), lambda i: (0, 0)),
        ],
        out_specs=pl.BlockSpec((_MV_CHUNK,), lambda i: (i,)),
        out_shape=jax.ShapeDtypeStruct((V,), jnp.float32),
    )(emb_table, W1, W2)

    info = plsc.get_sparse_core_info()
    nw = info.num_cores * info.num_subcores
    seqs_per_w = B // nw
    w3_pad = (S // 16 + 1) * 16
    w3_flat = jnp.zeros((w3_pad,), jnp.float32).at[:S].set(W3[:, 0])
    ids_pad = jnp.pad(input_ids, ((0, 0), (0, w3_pad - S)))
    t = _make_gather_fc3(nw, seqs_per_w, S, w3_pad)(ids_pad, p, w3_flat)

    out = pl.pallas_call(
        _head_body,
        in_specs=[
            pl.BlockSpec((B, 16), lambda: (0, 0)),
            pl.BlockSpec((S, 1), lambda: (0, 0)),
            pl.BlockSpec((1, H), lambda: (0, 0)),
            pl.BlockSpec((H, 1), lambda: (0, 0)),
            pl.BlockSpec((1, 1), lambda: (0, 0)),
            pl.BlockSpec((1, 1), lambda: (0, 0)),
        ],
        out_specs=pl.BlockSpec((B, 1), lambda: (0, 0)),
        out_shape=jax.ShapeDtypeStruct((B, 1), jnp.float32),
    )(t, W3, b1.reshape(1, H), W2, b2.reshape(1, 1), b3.reshape(1, 1))
    return out


# speedup vs baseline: 1.3227x; 1.3227x over previous
import functools

import jax
import jax.numpy as jnp
from jax import lax
from jax.experimental import pallas as pl
from jax.experimental.pallas import tpu as pltpu
from jax.experimental.pallas import tpu_sc as plsc

_LANES = 128
_MV_CHUNK = 32768


def _matvec_body(tab_ref, w1_ref, w2_ref, out_ref):
    v = jnp.dot(w1_ref[...], w2_ref[...], preferred_element_type=jnp.float32)
    acc = lax.dot_general(v.T, tab_ref[...], (((1,), (1,)), ((), ())),
                          preferred_element_type=jnp.float32)
    out_ref[...] = acc[0]


def _head_body(t_ref, w3_ref, b1_ref, w2_ref, b2_ref, b3_ref, out_ref):
    c = jnp.dot(b1_ref[...], w2_ref[...], preferred_element_type=jnp.float32)
    const = (c[0, 0] + b2_ref[0, 0]) * jnp.sum(w3_ref[...]) + b3_ref[0, 0]
    acc = jnp.dot(t_ref[...], w3_ref[...], preferred_element_type=jnp.float32)
    out_ref[...] = jax.nn.sigmoid(acc + const)


def _make_gather(num_workers, rows, table_size):
    nc = plsc.get_sparse_core_info().num_cores
    mesh = plsc.VectorSubcoreMesh(core_axis_name="c", subcore_axis_name="s")

    @functools.partial(
        pl.kernel,
        mesh=mesh,
        out_type=jax.ShapeDtypeStruct((num_workers, rows, _LANES), jnp.float32),
        scratch_types=[
            pltpu.VMEM((rows, _LANES), jnp.int32),
            pltpu.VMEM((rows, _LANES), jnp.float32),
            pltpu.SemaphoreType.DMA,
        ],
    )
    def gather_kernel(ids_hbm, p_hbm, out_hbm, idx_v, val_v, sem):
        wid = lax.axis_index("s") * nc + lax.axis_index("c")
        pltpu.sync_copy(ids_hbm.at[wid], idx_v)

        def fire(j, carry):
            pltpu.async_copy(p_hbm.at[idx_v.at[j]], val_v.at[j], sem)
            return carry

        lax.fori_loop(0, rows, fire, 0, unroll=False)

        def drain(j, carry):
            pltpu.make_async_copy(p_hbm.at[idx_v.at[j]], val_v.at[j], sem).wait()
            return carry

        lax.fori_loop(0, rows, drain, 0, unroll=False)
        pltpu.sync_copy(val_v, out_hbm.at[wid])

    return gather_kernel


def kernel(input_ids, emb_table, W1, b1, W2, b2, W3, b3):
    B, S = input_ids.shape
    V, D = emb_table.shape
    H = W1.shape[1]

    grid = (V + _MV_CHUNK - 1) // _MV_CHUNK
    p = pl.pallas_call(
        _matvec_body,
        grid=(grid,),
        in_specs=[
            pl.BlockSpec((_MV_CHUNK, D), lambda i: (i, 0)),
            pl.BlockSpec((D, H), lambda i: (0, 0)),
            pl.BlockSpec((H, 1), lambda i: (0, 0)),
        ],
        out_specs=pl.BlockSpec((_MV_CHUNK,), lambda i: (i,)),
        out_shape=jax.ShapeDtypeStruct((V,), jnp.float32),
    )(emb_table, W1, W2)

    info = plsc.get_sparse_core_info()
    nw = info.num_cores * info.num_subcores
    total = B * S
    rows = total // (nw * _LANES)
    ids3 = input_ids.reshape(nw, rows, _LANES)
    t = _make_gather(nw, rows, V)(ids3, p)
    t = t.reshape(B, S)

    out = pl.pallas_call(
        _head_body,
        in_specs=[
            pl.BlockSpec((B, S), lambda: (0, 0)),
            pl.BlockSpec((S, 1), lambda: (0, 0)),
            pl.BlockSpec((1, H), lambda: (0, 0)),
            pl.BlockSpec((H, 1), lambda: (0, 0)),
            pl.BlockSpec((1, 1), lambda: (0, 0)),
            pl.BlockSpec((1, 1), lambda: (0, 0)),
        ],
        out_specs=pl.BlockSpec((B, 1), lambda: (0, 0)),
        out_shape=jax.ShapeDtypeStruct((B, 1), jnp.float32),
    )(t, W3, b1.reshape(1, H), W2, b2.reshape(1, 1), b3.reshape(1, 1))
    return out
